# baseline (speedup 1.0000x reference)
import functools
import math

import jax
import jax.numpy as jnp
from jax import lax
from jax.experimental import pallas as pl
from jax.experimental.pallas import tpu as pltpu
from jax.experimental.pallas import tpu_sc as plsc

N = 10000
E = 320000
D = 128
NSH = 15
NTP = 8
DMSG = D + NTP * NSH
DOUT = 192
NSPECIES = 5
NBASIS = 8
HID = 64
AVG = 32.0

NC = 2
NS = 16
NW = NC * NS
EW = E // NW
CH = 80
KC = EW // CH
SPT = N // NS
C1W = DOUT - D
ET = E // NS
KC2 = ET // CH

GKG = 5
GKS = 2


def _gather_body(tab_hbm, idx_hbm, out_hbm, idx_v, rows_v, gsem, wsem):
    c = lax.axis_index("c")
    s = lax.axis_index("s")
    wid = c * NS + s
    pltpu.sync_copy(idx_hbm.at[wid], idx_v)
    ngroups = KC // GKG

    def grp(g, carry):
        p = lax.rem(g, 2) * GKG

        @pl.when(g >= 2)
        def _():
            for b in range(GKG):
                pltpu.make_async_copy(
                    rows_v.at[p + b], out_hbm.at[pl.ds(0, CH)], wsem).wait()

        for b in range(GKG):
            i = g * GKG + b
            pltpu.async_copy(tab_hbm.at[idx_v.at[i]], rows_v.at[p + b], gsem)
        for b in range(GKG):
            i = g * GKG + b
            pltpu.make_async_copy(
                tab_hbm.at[idx_v.at[i]], rows_v.at[p + b], gsem).wait()
        for b in range(GKG):
            i = g * GKG + b
            pltpu.async_copy(rows_v.at[p + b],
                             out_hbm.at[pl.ds(wid * EW + i * CH, CH)], wsem)
        return carry

    lax.fori_loop(0, ngroups, grp, 0)
    for b in range(2 * GKG):
        pltpu.make_async_copy(
            rows_v.at[b], out_hbm.at[pl.ds(0, CH)], wsem).wait()


@functools.cache
def _sc_kernels():
    mesh = plsc.VectorSubcoreMesh(core_axis_name="c", subcore_axis_name="s")
    gather = pl.kernel(
        _gather_body,
        out_type=jax.ShapeDtypeStruct((E, D), jnp.float32),
        mesh=mesh,
        compiler_params=pltpu.CompilerParams(use_tc_tiling_on_sc=False),
        scratch_types=[
            pltpu.VMEM((KC, CH), jnp.int32),
            pltpu.VMEM((2 * GKG, CH, D), jnp.float32),
            pltpu.SemaphoreType.DMA,
            pltpu.SemaphoreType.DMA,
        ],
    )
    scatter = pl.kernel(
        _scatter_body,
        out_type=jax.ShapeDtypeStruct((NC * N, D), jnp.float32),
        mesh=mesh,
        compiler_params=pltpu.CompilerParams(use_tc_tiling_on_sc=False),
        scratch_types=[
            pltpu.VMEM((2 * GKS, CH), jnp.int32),
            pltpu.VMEM((2 * GKS, CH, D), jnp.float32),
            pltpu.VMEM_SHARED((N, D), jnp.float32),
            pltpu.SemaphoreType.DMA,
            pltpu.SemaphoreType.DMA,
            pltpu.SemaphoreType.DMA,
        ],
    )
    return gather, scatter


def _scatter_body(y0_hbm, y1_hbm, idx_hbm, zeros_hbm, out_hbm, idx_r, rows_v,
                  acc_sh, isem, rsem, asem):
    c = lax.axis_index("c")
    s = lax.axis_index("s")
    pltpu.sync_copy(zeros_hbm, acc_sh.at[pl.ds(s * SPT, SPT)])
    plsc.subcore_barrier()

    ngroups = KC2 // GKS

    def grp(g, carry):
        p = lax.rem(g, 2) * GKS

        @pl.when(g >= 2)
        def _():
            for b in range(GKS):
                pltpu.make_async_copy(
                    rows_v.at[p + b], acc_sh.at[idx_r.at[0]], asem).wait()

        for b in range(GKS):
            i = g * GKS + b
            pltpu.async_copy(idx_hbm.at[s, i], idx_r.at[p + b], isem)

            @pl.when(c == 0)
            def _():
                pltpu.async_copy(y0_hbm.at[pl.ds(s * ET + i * CH, CH)],
                                 rows_v.at[p + b], rsem)

            @pl.when(c == 1)
            def _():
                pltpu.async_copy(y1_hbm.at[pl.ds(s * ET + i * CH, CH)],
                                 rows_v.at[p + b], rsem)
        for b in range(GKS):
            i = g * GKS + b
            pltpu.make_async_copy(
                idx_hbm.at[s, i], idx_r.at[p + b], isem).wait()
            pltpu.make_async_copy(
                y0_hbm.at[pl.ds(s * ET + i * CH, CH)],
                rows_v.at[p + b], rsem).wait()
        for b in range(GKS):
            pltpu.async_copy(rows_v.at[p + b], acc_sh.at[idx_r.at[p + b]],
                             asem, add=True)
        return carry

    lax.fori_loop(0, ngroups, grp, 0)
    for b in range(2 * GKS):
        pltpu.make_async_copy(
            rows_v.at[b], acc_sh.at[idx_r.at[0]], asem).wait()
    plsc.subcore_barrier()
    pltpu.sync_copy(
        acc_sh.at[pl.ds(s * SPT, SPT)],
        out_hbm.at[pl.ds(c * N + s * SPT, SPT)],
    )


_EB = 2560


def _edge_kernel(vtref, gref, w1, w2, w3, wd0, wd1, y0ref, y1ref):
    vt = vtref[...]
    x = vt[0:1, :]
    y = vt[1:2, :]
    z = vt[2:3, :]
    length = jnp.sqrt(x * x + y * y + z * z)
    safe = jnp.where(length == 0.0, 1.0, length)
    inv = 1.0 / safe
    ux, uy, uz = x * inv, y * inv, z * inv

    s3 = math.sqrt(3.0)
    s15 = math.sqrt(15.0)
    s5h = math.sqrt(5.0) / 2.0
    c1 = math.sqrt(35.0 / 8.0)
    c2 = math.sqrt(105.0)
    c3 = math.sqrt(21.0 / 8.0)
    c4 = math.sqrt(7.0) / 2.0
    zz = uz * uz
    shT = jnp.concatenate([
        s3 * ux, s3 * uy, s3 * uz,
        s15 * ux * uy,
        s15 * uy * uz,
        s5h * (3.0 * zz - 1.0),
        s15 * ux * uz,
        (s15 / 2.0) * (ux * ux - uy * uy),
        c1 * uy * (3.0 * ux * ux - uy * uy),
        c2 * ux * uy * uz,
        c3 * uy * (5.0 * zz - 1.0),
        c4 * uz * (5.0 * zz - 3.0),
        c3 * ux * (5.0 * zz - 1.0),
        (c2 / 2.0) * uz * (ux * ux - uy * uy),
        c1 * ux * (ux * ux - 3.0 * uy * uy),
    ], axis=0)

    kcol = (lax.broadcasted_iota(jnp.int32, (NBASIS, 1), 0) + 1).astype(jnp.float32)
    basisT = math.sqrt(2.0) * jnp.sin(jnp.pi * kcol * length) * inv
    stackT = jnp.concatenate(
        [shT, basisT, jnp.zeros((1, _EB), jnp.float32)], axis=0)
    st = stackT.T
    sh = st[:, :NSH]
    basis = st[:, NSH:NSH + NBASIS]

    h = jax.nn.silu(jnp.dot(basis, w1[...], preferred_element_type=jnp.float32))
    h = jax.nn.silu(jnp.dot(h, w2[...], preferred_element_type=jnp.float32))
    mix = jnp.dot(h.astype(jnp.bfloat16), w3[...].astype(jnp.bfloat16),
                  preferred_element_type=jnp.float32)

    msg = gref[...]
    rr = lax.broadcasted_iota(jnp.int32, (NTP, NTP * NSH), 0)
    rc = lax.broadcasted_iota(jnp.int32, (NTP, NTP * NSH), 1)
    Rm = (rc // NSH == rr).astype(jnp.float32)
    sr = lax.broadcasted_iota(jnp.int32, (NSH, NTP * NSH), 0)
    sc = lax.broadcasted_iota(jnp.int32, (NSH, NTP * NSH), 1)
    Sm = (sc % NSH == sr).astype(jnp.float32)
    tp = (jnp.dot(msg[:, :NTP], Rm, preferred_element_type=jnp.float32)
          * jnp.dot(sh, Sm, preferred_element_type=jnp.float32))

    messages = (jnp.concatenate([msg, tp], axis=1) * mix).astype(jnp.bfloat16)
    sc = 1.0 / math.sqrt(AVG)
    y0ref[...] = jnp.dot(messages, wd0[...].astype(jnp.bfloat16),
                         preferred_element_type=jnp.float32) * sc
    y1ref[...] = jnp.dot(messages, wd1[...].astype(jnp.bfloat16),
                         preferred_element_type=jnp.float32) * sc


_edge_call = pl.pallas_call(
    _edge_kernel,
    grid=(E // _EB,),
    in_specs=[
        pl.BlockSpec((3, _EB), lambda i: (0, i)),
        pl.BlockSpec((_EB, D), lambda i: (i, 0)),
        pl.BlockSpec((NBASIS, HID), lambda i: (0, 0)),
        pl.BlockSpec((HID, HID), lambda i: (0, 0)),
        pl.BlockSpec((HID, DMSG), lambda i: (0, 0)),
        pl.BlockSpec((DMSG, D), lambda i: (0, 0)),
        pl.BlockSpec((DMSG, D), lambda i: (0, 0)),
    ],
    out_specs=[
        pl.BlockSpec((_EB, D), lambda i: (i, 0)),
        pl.BlockSpec((_EB, D), lambda i: (i, 0)),
    ],
    out_shape=[
        jax.ShapeDtypeStruct((E, D), jnp.float32),
        jax.ShapeDtypeStruct((E, D), jnp.float32),
    ],
)


def _up_kernel(nfr, wup, outr):
    outr[...] = jnp.dot(nfr[...], wup[...], preferred_element_type=jnp.float32)


_up_call = pl.pallas_call(
    _up_kernel,
    grid=(10,),
    in_specs=[
        pl.BlockSpec((1000, D), lambda i: (i, 0)),
        pl.BlockSpec((D, D), lambda i: (0, 0)),
    ],
    out_specs=pl.BlockSpec((1000, D), lambda i: (i, 0)),
    out_shape=jax.ShapeDtypeStruct((N, D), jnp.float32),
)


_NB = 1000


def _node_kernel(a0, a1, nfr, spr, wsk, outr):
    nf = nfr[...]
    sp = spr[...]
    skip = jnp.zeros((_NB, DOUT), jnp.float32)
    for sidx in range(NSPECIES):
        m = (sp == sidx).astype(jnp.float32)
        skip = skip + m * jnp.dot(nf, wsk[sidx],
                                  preferred_element_type=jnp.float32)
    acc = jnp.concatenate([a0[...], a1[..., :C1W]], axis=1) + skip
    scal = jax.nn.silu(acc[:, :HID])
    gates = jax.nn.silu(acc[:, HID:2 * HID])
    vec = acc[:, 2 * HID:] * gates
    outr[...] = jnp.concatenate([scal, vec], axis=1)


_node_call = pl.pallas_call(
    _node_kernel,
    grid=(N // _NB,),
    in_specs=[
        pl.BlockSpec((_NB, D), lambda i: (i, 0)),
        pl.BlockSpec((_NB, D), lambda i: (i + N // _NB, 0)),
        pl.BlockSpec((_NB, D), lambda i: (i, 0)),
        pl.BlockSpec((_NB, 1), lambda i: (i, 0)),
        pl.BlockSpec((NSPECIES, D, DOUT), lambda i: (0, 0, 0)),
    ],
    out_specs=pl.BlockSpec((_NB, D), lambda i: (i, 0)),
    out_shape=jax.ShapeDtypeStruct((N, D), jnp.float32),
)


def kernel(vectors, node_feats, node_specie, senders, receivers,
           W_up, W1, W2, W3, W_skip, W_down):
    senders2 = senders.astype(jnp.int32).reshape(NW, KC, CH)
    receivers2 = receivers.astype(jnp.int32).reshape(NS, KC2, CH)

    _gather, _scatter = _sc_kernels()
    up = _up_call(node_feats, W_up)
    gathered = _gather(up, senders2)
    wd1p = jnp.concatenate(
        [W_down[:, D:], jnp.zeros((DMSG, D - C1W), jnp.float32)], axis=1)
    y0, y1 = _edge_call(vectors.T, gathered, W1, W2, W3,
                        W_down[:, :D], wd1p)
    zeros = jnp.zeros((SPT, D), jnp.float32)
    accs = _scatter(y0, y1, receivers2, zeros)
    out = _node_call(accs, accs, node_feats,
                     node_specie.astype(jnp.int32).reshape(N, 1), W_skip)
    return out


# speedup vs baseline: 1.1345x; 1.1345x over previous
import functools
import math

import jax
import jax.numpy as jnp
from jax import lax
from jax.experimental import pallas as pl
from jax.experimental.pallas import tpu as pltpu
from jax.experimental.pallas import tpu_sc as plsc

N = 10000
E = 320000
D = 128
NSH = 15
NTP = 8
DMSG = D + NTP * NSH
DOUT = 192
NSPECIES = 5
NBASIS = 8
HID = 64
AVG = 32.0

NC = 2
NS = 16
NW = NC * NS
NH = 2
EH = E // NH
EW = EH // NW
CHG = 100
KC = EW // CHG
SPT = N // NS
C1W = DOUT - D
ET = EH // NS
CHS = 40
KC2 = ET // CHS

GKG = 5
GKS = 2


def _gather_body(tab_hbm, idx_hbm, out_hbm, idx_r, rows_v, isem, gsem, wsem):
    c = lax.axis_index("c")
    s = lax.axis_index("s")
    wid = c * NS + s
    ngroups = KC // GKG

    def grp(g, carry):
        p = lax.rem(g, 2) * GKG

        @pl.when(g >= 2)
        def _():
            for b in range(GKG):
                pltpu.make_async_copy(
                    rows_v.at[p + b], out_hbm.at[pl.ds(0, CHG)], wsem).wait()

        for b in range(GKG):
            i = g * GKG + b
            pltpu.async_copy(idx_hbm.at[wid, i], idx_r.at[p + b], isem)
        for b in range(GKG):
            i = g * GKG + b
            pltpu.make_async_copy(
                idx_hbm.at[wid, i], idx_r.at[p + b], isem).wait()
            pltpu.async_copy(tab_hbm.at[idx_r.at[p + b]], rows_v.at[p + b],
                             gsem)
        for b in range(GKG):
            pltpu.make_async_copy(
                tab_hbm.at[idx_r.at[p + b]], rows_v.at[p + b], gsem).wait()
        for b in range(GKG):
            i = g * GKG + b
            pltpu.async_copy(rows_v.at[p + b],
                             out_hbm.at[pl.ds(wid * EW + i * CHG, CHG)], wsem)
        return carry

    lax.fori_loop(0, ngroups, grp, 0)
    for b in range(2 * GKG):
        pltpu.make_async_copy(
            rows_v.at[b], out_hbm.at[pl.ds(0, CHG)], wsem).wait()


@functools.cache
def _sc_kernels():
    mesh = plsc.VectorSubcoreMesh(core_axis_name="c", subcore_axis_name="s")
    gather = pl.kernel(
        _gather_body,
        out_type=jax.ShapeDtypeStruct((EH, D), jnp.float32),
        mesh=mesh,
        compiler_params=pltpu.CompilerParams(use_tc_tiling_on_sc=False),
        scratch_types=[
            pltpu.VMEM((2 * GKG, CHG), jnp.int32),
            pltpu.VMEM((2 * GKG, CHG, D), jnp.float32),
            pltpu.SemaphoreType.DMA,
            pltpu.SemaphoreType.DMA,
            pltpu.SemaphoreType.DMA,
        ],
    )
    scatter = pl.kernel(
        _scatter_body,
        out_type=jax.ShapeDtypeStruct((NC * N, D), jnp.float32),
        mesh=mesh,
        compiler_params=pltpu.CompilerParams(use_tc_tiling_on_sc=False),
        scratch_types=[
            pltpu.VMEM((2 * GKS, CHS), jnp.int32),
            pltpu.VMEM((2 * GKS, CHS, D), jnp.float32),
            pltpu.VMEM_SHARED((N, D), jnp.float32),
            pltpu.SemaphoreType.DMA,
            pltpu.SemaphoreType.DMA,
            pltpu.SemaphoreType.DMA,
        ],
    )
    return gather, scatter


def _scatter_body(y0_hbm, y1_hbm, idx_hbm, zeros_hbm, out_hbm, idx_r, rows_v,
                  acc_sh, isem, rsem, asem):
    c = lax.axis_index("c")
    s = lax.axis_index("s")
    pltpu.sync_copy(zeros_hbm, acc_sh.at[pl.ds(s * SPT, SPT)])
    plsc.subcore_barrier()

    ngroups = KC2 // GKS

    def grp(g, carry):
        p = lax.rem(g, 2) * GKS

        @pl.when(g >= 2)
        def _():
            for b in range(GKS):
                pltpu.make_async_copy(
                    rows_v.at[p + b], acc_sh.at[idx_r.at[0]], asem).wait()

        for b in range(GKS):
            i = g * GKS + b
            pltpu.async_copy(idx_hbm.at[s, i], idx_r.at[p + b], isem)

            @pl.when(c == 0)
            def _():
                pltpu.async_copy(y0_hbm.at[pl.ds(s * ET + i * CHS, CHS)],
                                 rows_v.at[p + b], rsem)

            @pl.when(c == 1)
            def _():
                pltpu.async_copy(y1_hbm.at[pl.ds(s * ET + i * CHS, CHS)],
                                 rows_v.at[p + b], rsem)
        for b in range(GKS):
            i = g * GKS + b
            pltpu.make_async_copy(
                idx_hbm.at[s, i], idx_r.at[p + b], isem).wait()
            pltpu.make_async_copy(
                y0_hbm.at[pl.ds(s * ET + i * CHS, CHS)],
                rows_v.at[p + b], rsem).wait()
        for b in range(GKS):
            pltpu.async_copy(rows_v.at[p + b], acc_sh.at[idx_r.at[p + b]],
                             asem, add=True)
        return carry

    lax.fori_loop(0, ngroups, grp, 0)
    for b in range(2 * GKS):
        pltpu.make_async_copy(
            rows_v.at[b], acc_sh.at[idx_r.at[0]], asem).wait()
    plsc.subcore_barrier()
    pltpu.sync_copy(
        acc_sh.at[pl.ds(s * SPT, SPT)],
        out_hbm.at[pl.ds(c * N + s * SPT, SPT)],
    )


_EB = 6400


def _edge_kernel(vtref, gref, w1, w2, w3, wd0, wd1, y0ref, y1ref):
    vt = vtref[...]
    x = vt[0:1, :]
    y = vt[1:2, :]
    z = vt[2:3, :]
    length = jnp.sqrt(x * x + y * y + z * z)
    safe = jnp.where(length == 0.0, 1.0, length)
    inv = 1.0 / safe
    ux, uy, uz = x * inv, y * inv, z * inv

    s3 = math.sqrt(3.0)
    s15 = math.sqrt(15.0)
    s5h = math.sqrt(5.0) / 2.0
    c1 = math.sqrt(35.0 / 8.0)
    c2 = math.sqrt(105.0)
    c3 = math.sqrt(21.0 / 8.0)
    c4 = math.sqrt(7.0) / 2.0
    zz = uz * uz
    shT = jnp.concatenate([
        s3 * ux, s3 * uy, s3 * uz,
        s15 * ux * uy,
        s15 * uy * uz,
        s5h * (3.0 * zz - 1.0),
        s15 * ux * uz,
        (s15 / 2.0) * (ux * ux - uy * uy),
        c1 * uy * (3.0 * ux * ux - uy * uy),
        c2 * ux * uy * uz,
        c3 * uy * (5.0 * zz - 1.0),
        c4 * uz * (5.0 * zz - 3.0),
        c3 * ux * (5.0 * zz - 1.0),
        (c2 / 2.0) * uz * (ux * ux - uy * uy),
        c1 * ux * (ux * ux - 3.0 * uy * uy),
    ], axis=0)

    kcol = (lax.broadcasted_iota(jnp.int32, (NBASIS, 1), 0) + 1).astype(jnp.float32)
    basisT = math.sqrt(2.0) * jnp.sin(jnp.pi * kcol * length) * inv
    stackT = jnp.concatenate(
        [shT, basisT, jnp.zeros((1, _EB), jnp.float32)], axis=0)
    st = stackT.T
    sh = st[:, :NSH]
    basis = st[:, NSH:NSH + NBASIS]

    h = jax.nn.silu(jnp.dot(basis, w1[...], preferred_element_type=jnp.float32))
    h = jax.nn.silu(jnp.dot(h, w2[...], preferred_element_type=jnp.float32))
    mix = jnp.dot(h.astype(jnp.bfloat16), w3[...].astype(jnp.bfloat16),
                  preferred_element_type=jnp.float32)

    msg = gref[...]
    rr = lax.broadcasted_iota(jnp.int32, (NTP, NTP * NSH), 0)
    rc = lax.broadcasted_iota(jnp.int32, (NTP, NTP * NSH), 1)
    Rm = (rc // NSH == rr).astype(jnp.float32)
    sr = lax.broadcasted_iota(jnp.int32, (NSH, NTP * NSH), 0)
    sc = lax.broadcasted_iota(jnp.int32, (NSH, NTP * NSH), 1)
    Sm = (sc % NSH == sr).astype(jnp.float32)
    tp = (jnp.dot(msg[:, :NTP], Rm, preferred_element_type=jnp.float32)
          * jnp.dot(sh, Sm, preferred_element_type=jnp.float32))

    messages = (jnp.concatenate([msg, tp], axis=1) * mix).astype(jnp.bfloat16)
    sc = 1.0 / math.sqrt(AVG)
    y0ref[...] = jnp.dot(messages, wd0[...].astype(jnp.bfloat16),
                         preferred_element_type=jnp.float32) * sc
    y1ref[...] = jnp.dot(messages, wd1[...].astype(jnp.bfloat16),
                         preferred_element_type=jnp.float32) * sc


def _make_edge_call(h):
    off = h * (EH // _EB)
    return pl.pallas_call(
        _edge_kernel,
        grid=(EH // _EB,),
        in_specs=[
            pl.BlockSpec((3, _EB), lambda i: (0, i + off)),
            pl.BlockSpec((_EB, D), lambda i: (i, 0)),
            pl.BlockSpec((NBASIS, HID), lambda i: (0, 0)),
            pl.BlockSpec((HID, HID), lambda i: (0, 0)),
            pl.BlockSpec((HID, DMSG), lambda i: (0, 0)),
            pl.BlockSpec((DMSG, D), lambda i: (0, 0)),
            pl.BlockSpec((DMSG, D), lambda i: (0, 0)),
        ],
        out_specs=[
            pl.BlockSpec((_EB, D), lambda i: (i, 0)),
            pl.BlockSpec((_EB, D), lambda i: (i, 0)),
        ],
        out_shape=[
            jax.ShapeDtypeStruct((EH, D), jnp.float32),
            jax.ShapeDtypeStruct((EH, D), jnp.float32),
        ],
    )


_edge_calls = [_make_edge_call(0), _make_edge_call(1)]


def _up_kernel(nfr, wup, outr):
    outr[...] = jnp.dot(nfr[...], wup[...], preferred_element_type=jnp.float32)


_up_call = pl.pallas_call(
    _up_kernel,
    grid=(10,),
    in_specs=[
        pl.BlockSpec((1000, D), lambda i: (i, 0)),
        pl.BlockSpec((D, D), lambda i: (0, 0)),
    ],
    out_specs=pl.BlockSpec((1000, D), lambda i: (i, 0)),
    out_shape=jax.ShapeDtypeStruct((N, D), jnp.float32),
)


_NB = 1000


def _node_kernel(a0a, a1a, a0b, a1b, nfr, spr, wsk, outr):
    nf = nfr[...]
    sp = spr[...]
    skip = jnp.zeros((_NB, DOUT), jnp.float32)
    for sidx in range(NSPECIES):
        m = (sp == sidx).astype(jnp.float32)
        skip = skip + m * jnp.dot(nf, wsk[sidx],
                                  preferred_element_type=jnp.float32)
    a0 = a0a[...] + a0b[...]
    a1 = a1a[...] + a1b[...]
    acc = jnp.concatenate([a0, a1[:, :C1W]], axis=1) + skip
    scal = jax.nn.silu(acc[:, :HID])
    gates = jax.nn.silu(acc[:, HID:2 * HID])
    vec = acc[:, 2 * HID:] * gates
    outr[...] = jnp.concatenate([scal, vec], axis=1)


_node_call = pl.pallas_call(
    _node_kernel,
    grid=(N // _NB,),
    in_specs=[
        pl.BlockSpec((_NB, D), lambda i: (i, 0)),
        pl.BlockSpec((_NB, D), lambda i: (i + N // _NB, 0)),
        pl.BlockSpec((_NB, D), lambda i: (i, 0)),
        pl.BlockSpec((_NB, D), lambda i: (i + N // _NB, 0)),
        pl.BlockSpec((_NB, D), lambda i: (i, 0)),
        pl.BlockSpec((_NB, 1), lambda i: (i, 0)),
        pl.BlockSpec((NSPECIES, D, DOUT), lambda i: (0, 0, 0)),
    ],
    out_specs=pl.BlockSpec((_NB, D), lambda i: (i, 0)),
    out_shape=jax.ShapeDtypeStruct((N, D), jnp.float32),
)


def kernel(vectors, node_feats, node_specie, senders, receivers,
           W_up, W1, W2, W3, W_skip, W_down):
    senders2 = senders.astype(jnp.int32).reshape(NH, NW, KC, CHG)
    receivers2 = receivers.astype(jnp.int32).reshape(NH, NS, KC2, CHS)

    _gather, _scatter = _sc_kernels()
    up = _up_call(node_feats, W_up)
    wd0 = W_down[:, :D]
    wd1p = jnp.concatenate(
        [W_down[:, D:], jnp.zeros((DMSG, D - C1W), jnp.float32)], axis=1)
    zeros = jnp.zeros((SPT, D), jnp.float32)
    vt = vectors.T

    accs = []
    for h in range(NH):
        gathered = _gather(up, senders2[h])
        y0, y1 = _edge_calls[h](vt, gathered, W1, W2, W3, wd0, wd1p)
        accs.append(_scatter(y0, y1, receivers2[h], zeros))
    out = _node_call(accs[0], accs[0], accs[1], accs[1], node_feats,
                     node_specie.astype(jnp.int32).reshape(N, 1), W_skip)
    return out


# speedup vs baseline: 1.1351x; 1.0005x over previous
import functools
import math

import jax
import jax.numpy as jnp
from jax import lax
from jax.experimental import pallas as pl
from jax.experimental.pallas import tpu as pltpu
from jax.experimental.pallas import tpu_sc as plsc

N = 10000
E = 320000
D = 128
NSH = 15
NTP = 8
DMSG = D + NTP * NSH
DOUT = 192
NSPECIES = 5
NBASIS = 8
HID = 64
AVG = 32.0

NC = 2
NS = 16
NW = NC * NS
NH = 2
EH = E // NH
EW = EH // NW
CHG = 100
KC = EW // CHG
SPT = N // NS
C1W = DOUT - D
ET = EH // NS
CHS = 40
KC2 = ET // CHS

GKG = 5
GKS = 2
RS = 4


def _gather_body(tab_hbm, idx_hbm, out_hbm, idx_r, rows_v, isem, gsem, wsem):
    c = lax.axis_index("c")
    s = lax.axis_index("s")
    wid = c * NS + s
    ngroups = KC // GKG

    def grp(g, carry):
        p = lax.rem(g, 2) * GKG

        @pl.when(g >= 2)
        def _():
            for b in range(GKG):
                pltpu.make_async_copy(
                    rows_v.at[p + b], out_hbm.at[pl.ds(0, CHG)], wsem).wait()

        for b in range(GKG):
            i = g * GKG + b
            pltpu.async_copy(idx_hbm.at[wid, i], idx_r.at[p + b], isem)
        for b in range(GKG):
            i = g * GKG + b
            pltpu.make_async_copy(
                idx_hbm.at[wid, i], idx_r.at[p + b], isem).wait()
            pltpu.async_copy(tab_hbm.at[idx_r.at[p + b]], rows_v.at[p + b],
                             gsem)
        for b in range(GKG):
            pltpu.make_async_copy(
                tab_hbm.at[idx_r.at[p + b]], rows_v.at[p + b], gsem).wait()
        for b in range(GKG):
            i = g * GKG + b
            pltpu.async_copy(rows_v.at[p + b],
                             out_hbm.at[pl.ds(wid * EW + i * CHG, CHG)], wsem)
        return carry

    lax.fori_loop(0, ngroups, grp, 0)
    for b in range(2 * GKG):
        pltpu.make_async_copy(
            rows_v.at[b], out_hbm.at[pl.ds(0, CHG)], wsem).wait()


@functools.cache
def _sc_kernels():
    mesh = plsc.VectorSubcoreMesh(core_axis_name="c", subcore_axis_name="s")
    gather = pl.kernel(
        _gather_body,
        out_type=jax.ShapeDtypeStruct((EH, D), jnp.float32),
        mesh=mesh,
        compiler_params=pltpu.CompilerParams(use_tc_tiling_on_sc=False),
        scratch_types=[
            pltpu.VMEM((2 * GKG, CHG), jnp.int32),
            pltpu.VMEM((2 * GKG, CHG, D), jnp.float32),
            pltpu.SemaphoreType.DMA,
            pltpu.SemaphoreType.DMA,
            pltpu.SemaphoreType.DMA,
        ],
    )
    scatter = pl.kernel(
        _scatter_body,
        out_type=jax.ShapeDtypeStruct((NC * N, D), jnp.float32),
        mesh=mesh,
        compiler_params=pltpu.CompilerParams(use_tc_tiling_on_sc=False),
        scratch_types=[
            pltpu.VMEM((RS * GKS, CHS), jnp.int32),
            pltpu.VMEM((RS * GKS, CHS, D), jnp.float32),
            pltpu.VMEM_SHARED((N, D), jnp.float32),
            pltpu.SemaphoreType.DMA,
            pltpu.SemaphoreType.DMA,
            pltpu.SemaphoreType.DMA,
        ],
    )
    return gather, scatter


def _scatter_body(y0_hbm, y1_hbm, idx_hbm, zeros_hbm, out_hbm, idx_r, rows_v,
                  acc_sh, isem, rsem, asem):
    c = lax.axis_index("c")
    s = lax.axis_index("s")
    pltpu.sync_copy(zeros_hbm, acc_sh.at[pl.ds(s * SPT, SPT)])
    plsc.subcore_barrier()

    ngroups = KC2 // GKS

    def grp(g, carry):
        p = lax.rem(g, RS) * GKS

        @pl.when(g >= RS)
        def _():
            for b in range(GKS):
                pltpu.make_async_copy(
                    rows_v.at[p + b], acc_sh.at[idx_r.at[0]], asem).wait()

        for b in range(GKS):
            i = g * GKS + b
            pltpu.async_copy(idx_hbm.at[s, i], idx_r.at[p + b], isem)

            @pl.when(c == 0)
            def _():
                pltpu.async_copy(y0_hbm.at[pl.ds(s * ET + i * CHS, CHS)],
                                 rows_v.at[p + b], rsem)

            @pl.when(c == 1)
            def _():
                pltpu.async_copy(y1_hbm.at[pl.ds(s * ET + i * CHS, CHS)],
                                 rows_v.at[p + b], rsem)
        for b in range(GKS):
            i = g * GKS + b
            pltpu.make_async_copy(
                idx_hbm.at[s, i], idx_r.at[p + b], isem).wait()
            pltpu.make_async_copy(
                y0_hbm.at[pl.ds(s * ET + i * CHS, CHS)],
                rows_v.at[p + b], rsem).wait()
        for b in range(GKS):
            pltpu.async_copy(rows_v.at[p + b], acc_sh.at[idx_r.at[p + b]],
                             asem, add=True)
        return carry

    lax.fori_loop(0, ngroups, grp, 0)
    for b in range(RS * GKS):
        pltpu.make_async_copy(
            rows_v.at[b], acc_sh.at[idx_r.at[0]], asem).wait()
    plsc.subcore_barrier()
    pltpu.sync_copy(
        acc_sh.at[pl.ds(s * SPT, SPT)],
        out_hbm.at[pl.ds(c * N + s * SPT, SPT)],
    )


_EB = 6400


def _edge_kernel(vtref, gref, w1, w2, w3, wd0, wd1, y0ref, y1ref):
    vt = vtref[...]
    x = vt[0:1, :]
    y = vt[1:2, :]
    z = vt[2:3, :]
    length = jnp.sqrt(x * x + y * y + z * z)
    safe = jnp.where(length == 0.0, 1.0, length)
    inv = 1.0 / safe
    ux, uy, uz = x * inv, y * inv, z * inv

    s3 = math.sqrt(3.0)
    s15 = math.sqrt(15.0)
    s5h = math.sqrt(5.0) / 2.0
    c1 = math.sqrt(35.0 / 8.0)
    c2 = math.sqrt(105.0)
    c3 = math.sqrt(21.0 / 8.0)
    c4 = math.sqrt(7.0) / 2.0
    zz = uz * uz
    shT = jnp.concatenate([
        s3 * ux, s3 * uy, s3 * uz,
        s15 * ux * uy,
        s15 * uy * uz,
        s5h * (3.0 * zz - 1.0),
        s15 * ux * uz,
        (s15 / 2.0) * (ux * ux - uy * uy),
        c1 * uy * (3.0 * ux * ux - uy * uy),
        c2 * ux * uy * uz,
        c3 * uy * (5.0 * zz - 1.0),
        c4 * uz * (5.0 * zz - 3.0),
        c3 * ux * (5.0 * zz - 1.0),
        (c2 / 2.0) * uz * (ux * ux - uy * uy),
        c1 * ux * (ux * ux - 3.0 * uy * uy),
    ], axis=0)

    kcol = (lax.broadcasted_iota(jnp.int32, (NBASIS, 1), 0) + 1).astype(jnp.float32)
    basisT = math.sqrt(2.0) * jnp.sin(jnp.pi * kcol * length) * inv
    stackT = jnp.concatenate(
        [shT, basisT, jnp.zeros((1, _EB), jnp.float32)], axis=0)
    st = stackT.T
    sh = st[:, :NSH]
    basis = st[:, NSH:NSH + NBASIS]

    h = jax.nn.silu(jnp.dot(basis, w1[...], preferred_element_type=jnp.float32))
    h = jax.nn.silu(jnp.dot(h, w2[...], preferred_element_type=jnp.float32))
    mix = jnp.dot(h.astype(jnp.bfloat16), w3[...].astype(jnp.bfloat16),
                  preferred_element_type=jnp.float32)

    msg = gref[...]
    rr = lax.broadcasted_iota(jnp.int32, (NTP, NTP * NSH), 0)
    rc = lax.broadcasted_iota(jnp.int32, (NTP, NTP * NSH), 1)
    Rm = (rc // NSH == rr).astype(jnp.float32)
    sr = lax.broadcasted_iota(jnp.int32, (NSH, NTP * NSH), 0)
    sc = lax.broadcasted_iota(jnp.int32, (NSH, NTP * NSH), 1)
    Sm = (sc % NSH == sr).astype(jnp.float32)
    tp = (jnp.dot(msg[:, :NTP], Rm, preferred_element_type=jnp.float32)
          * jnp.dot(sh, Sm, preferred_element_type=jnp.float32))

    messages = (jnp.concatenate([msg, tp], axis=1) * mix).astype(jnp.bfloat16)
    sc = 1.0 / math.sqrt(AVG)
    y0ref[...] = jnp.dot(messages, wd0[...].astype(jnp.bfloat16),
                         preferred_element_type=jnp.float32) * sc
    y1ref[...] = jnp.dot(messages, wd1[...].astype(jnp.bfloat16),
                         preferred_element_type=jnp.float32) * sc


def _make_edge_call(h):
    off = h * (EH // _EB)
    return pl.pallas_call(
        _edge_kernel,
        grid=(EH // _EB,),
        in_specs=[
            pl.BlockSpec((3, _EB), lambda i: (0, i + off)),
            pl.BlockSpec((_EB, D), lambda i: (i, 0)),
            pl.BlockSpec((NBASIS, HID), lambda i: (0, 0)),
            pl.BlockSpec((HID, HID), lambda i: (0, 0)),
            pl.BlockSpec((HID, DMSG), lambda i: (0, 0)),
            pl.BlockSpec((DMSG, D), lambda i: (0, 0)),
            pl.BlockSpec((DMSG, D), lambda i: (0, 0)),
        ],
        out_specs=[
            pl.BlockSpec((_EB, D), lambda i: (i, 0)),
            pl.BlockSpec((_EB, D), lambda i: (i, 0)),
        ],
        out_shape=[
            jax.ShapeDtypeStruct((EH, D), jnp.float32),
            jax.ShapeDtypeStruct((EH, D), jnp.float32),
        ],
    )


_edge_calls = [_make_edge_call(0), _make_edge_call(1)]


def _up_kernel(nfr, wup, outr):
    outr[...] = jnp.dot(nfr[...], wup[...], preferred_element_type=jnp.float32)


_up_call = pl.pallas_call(
    _up_kernel,
    grid=(10,),
    in_specs=[
        pl.BlockSpec((1000, D), lambda i: (i, 0)),
        pl.BlockSpec((D, D), lambda i: (0, 0)),
    ],
    out_specs=pl.BlockSpec((1000, D), lambda i: (i, 0)),
    out_shape=jax.ShapeDtypeStruct((N, D), jnp.float32),
)


_NB = 1000


def _node_kernel(a0a, a1a, a0b, a1b, nfr, spr, wsk, outr):
    nf = nfr[...]
    sp = spr[...]
    skip = jnp.zeros((_NB, DOUT), jnp.float32)
    for sidx in range(NSPECIES):
        m = (sp == sidx).astype(jnp.float32)
        skip = skip + m * jnp.dot(nf, wsk[sidx],
                                  preferred_element_type=jnp.float32)
    a0 = a0a[...] + a0b[...]
    a1 = a1a[...] + a1b[...]
    acc = jnp.concatenate([a0, a1[:, :C1W]], axis=1) + skip
    scal = jax.nn.silu(acc[:, :HID])
    gates = jax.nn.silu(acc[:, HID:2 * HID])
    vec = acc[:, 2 * HID:] * gates
    outr[...] = jnp.concatenate([scal, vec], axis=1)


_node_call = pl.pallas_call(
    _node_kernel,
    grid=(N // _NB,),
    in_specs=[
        pl.BlockSpec((_NB, D), lambda i: (i, 0)),
        pl.BlockSpec((_NB, D), lambda i: (i + N // _NB, 0)),
        pl.BlockSpec((_NB, D), lambda i: (i, 0)),
        pl.BlockSpec((_NB, D), lambda i: (i + N // _NB, 0)),
        pl.BlockSpec((_NB, D), lambda i: (i, 0)),
        pl.BlockSpec((_NB, 1), lambda i: (i, 0)),
        pl.BlockSpec((NSPECIES, D, DOUT), lambda i: (0, 0, 0)),
    ],
    out_specs=pl.BlockSpec((_NB, D), lambda i: (i, 0)),
    out_shape=jax.ShapeDtypeStruct((N, D), jnp.float32),
)


def kernel(vectors, node_feats, node_specie, senders, receivers,
           W_up, W1, W2, W3, W_skip, W_down):
    senders2 = senders.astype(jnp.int32).reshape(NH, NW, KC, CHG)
    receivers2 = receivers.astype(jnp.int32).reshape(NH, NS, KC2, CHS)

    _gather, _scatter = _sc_kernels()
    up = _up_call(node_feats, W_up)
    wd0 = W_down[:, :D]
    wd1p = jnp.concatenate(
        [W_down[:, D:], jnp.zeros((DMSG, D - C1W), jnp.float32)], axis=1)
    zeros = jnp.zeros((SPT, D), jnp.float32)
    vt = vectors.T

    accs = []
    for h in range(NH):
        gathered = _gather(up, senders2[h])
        y0, y1 = _edge_calls[h](vt, gathered, W1, W2, W3, wd0, wd1p)
        accs.append(_scatter(y0, y1, receivers2[h], zeros))
    out = _node_call(accs[0], accs[0], accs[1], accs[1], node_feats,
                     node_specie.astype(jnp.int32).reshape(N, 1), W_skip)
    return out
